# baseline (speedup 1.0000x reference)
import jax
import jax.numpy as jnp
from jax import lax
from jax.experimental import pallas as pl
from jax.experimental.pallas import tpu as pltpu

NUM_SEG = 64
BLOCK = 5000


def _pool_kernel(batch_ref, x_ref, W1_ref, b1_ref, W2_ref, b2_ref,
                 out_ref, acc_ref, m_ref, s_ref):
    i = pl.program_id(0)
    nblk = pl.num_programs(0)

    @pl.when(i == 0)
    def _init():
        acc_ref[...] = jnp.zeros_like(acc_ref)
        m_ref[...] = jnp.full_like(m_ref, -jnp.inf)
        s_ref[...] = jnp.zeros_like(s_ref)

    x = x_ref[...]
    h = jnp.tanh(jnp.dot(x, W1_ref[...],
                         preferred_element_type=jnp.float32) + b1_ref[...])
    logit = (jnp.sum(h * W2_ref[...], axis=1, keepdims=True)
             + b2_ref[0, 0])
    lt = logit.reshape(1, BLOCK)

    seg = batch_ref[...].reshape(1, BLOCK)
    row = lax.broadcasted_iota(jnp.int32, (NUM_SEG, BLOCK), 0)
    onehot = seg == row

    masked = jnp.where(onehot, lt, -jnp.inf)
    bmax = jnp.max(masked, axis=1, keepdims=True)
    m_old = m_ref[...]
    m_new = jnp.maximum(m_old, bmax)
    scale = jnp.where(m_old == m_new, 1.0, jnp.exp(m_old - m_new))
    m_ref[...] = m_new

    rowm = jnp.sum(jnp.where(onehot, m_new, 0.0), axis=0, keepdims=True)
    p = jnp.exp(lt - rowm)
    wp = jnp.where(onehot, p, 0.0)

    s_ref[...] = s_ref[...] * scale + jnp.sum(wp, axis=1, keepdims=True)
    contrib = jnp.dot(wp, x, preferred_element_type=jnp.float32)
    acc_ref[...] = acc_ref[...] * scale + contrib

    @pl.when(i == nblk - 1)
    def _fin():
        out_ref[...] = acc_ref[...] / (s_ref[...] + 1e-8)


def kernel(x, batch, W1, b1, W2, b2):
    n, in_dim = x.shape
    hidden = W1.shape[1]
    nblk = pl.cdiv(n, BLOCK)
    pad = nblk * BLOCK - n
    if pad:
        x = jnp.pad(x, ((0, pad), (0, 0)))
        batch = jnp.pad(batch, (0, pad), constant_values=NUM_SEG)
    batch3 = batch.reshape(nblk, 1, BLOCK)

    out = pl.pallas_call(
        _pool_kernel,
        grid=(nblk,),
        in_specs=[
            pl.BlockSpec((1, 1, BLOCK), lambda i: (i, 0, 0)),
            pl.BlockSpec((BLOCK, in_dim), lambda i: (i, 0)),
            pl.BlockSpec((in_dim, hidden), lambda i: (0, 0)),
            pl.BlockSpec((1, hidden), lambda i: (0, 0)),
            pl.BlockSpec((1, hidden), lambda i: (0, 0)),
            pl.BlockSpec((1, 1), lambda i: (0, 0)),
        ],
        out_specs=pl.BlockSpec((NUM_SEG, in_dim), lambda i: (0, 0)),
        out_shape=jax.ShapeDtypeStruct((NUM_SEG, in_dim), jnp.float32),
        scratch_shapes=[
            pltpu.VMEM((NUM_SEG, in_dim), jnp.float32),
            pltpu.VMEM((NUM_SEG, 1), jnp.float32),
            pltpu.VMEM((NUM_SEG, 1), jnp.float32),
        ],
    )(batch3, x, W1, b1.reshape(1, hidden), W2.reshape(1, hidden),
      b2.reshape(1, 1))
    return out
